# baseline (speedup 1.0000x reference)
import functools

import jax
import jax.numpy as jnp
from jax import lax
from jax.experimental import pallas as pl
from jax.experimental.pallas import tpu as pltpu
from jax.experimental.pallas import tpu_sc as plsc

_N = 10000
_E = 320000
_L = 128
_DE = 16
_NL = 4

_NPAD = 10240
_NB = 128
_EPAD = 327680
_NBATCH = _EPAD // _NB
_NC, _NS = 2, 16
_NW = _NC * _NS
_BPW = _NBATCH // _NW
_RPS = _NPAD // _NS

_NROWB = 1024
_EROWB = 2048


def _dot(a, b):
    return jax.lax.dot(a, b)


def _elu(x):
    return jnp.where(x > 0, x, jnp.exp(jnp.minimum(x, 0.0)) - 1.0)


def _mlp3_body(x_ref, w0, b0, w1, b1, w2, b2, o_ref, *, final_elu):
    h = _elu(_dot(x_ref[...], w0[...]) + b0[...])
    h = _elu(_dot(h, w1[...]) + b1[...])
    y = _dot(h, w2[...]) + b2[...]
    o_ref[...] = _elu(y) if final_elu else y


def _mlp3_call(x, w0, b0, w1, b1, w2, b2, final_elu):
    n = x.shape[0]
    grid = (n // _NROWB,)
    row = pl.BlockSpec((_NROWB, x.shape[1]), lambda i: (i, 0))
    full = lambda a: pl.BlockSpec(a.shape, lambda i: (0, 0))
    return pl.pallas_call(
        functools.partial(_mlp3_body, final_elu=final_elu),
        grid=grid,
        in_specs=[row, full(w0), full(b0), full(w1), full(b1), full(w2), full(b2)],
        out_specs=pl.BlockSpec((_NROWB, w2.shape[1]), lambda i: (i, 0)),
        out_shape=jax.ShapeDtypeStruct((n, w2.shape[1]), jnp.float32),
    )(x, w0, b0, w1, b1, w2, b2)


def _edge_body(td_ref, ts_ref, ea_ref, w0, b0, w1, b1, w2, b2, o_ref):
    feat = jnp.concatenate([td_ref[...], ts_ref[...], ea_ref[...]], axis=1)
    h1 = _elu(_dot(feat, w0[...]) + b0[...])
    h2 = _elu(_dot(h1, w1[...]) + b1[...])
    o_ref[...] = _dot(h2, w2[...]) + b2[...]


def _edge_call(td, ts, ea, w0, b0, w1, b1, w2, b2):
    grid = (_EPAD // _EROWB,)
    row = pl.BlockSpec((_EROWB, _L), lambda i: (i, 0))
    ear = pl.BlockSpec((_EROWB, _DE), lambda i: (i, 0))
    full = lambda a: pl.BlockSpec(a.shape, lambda i: (0, 0))
    return pl.pallas_call(
        _edge_body,
        grid=grid,
        in_specs=[row, row, ear, full(w0), full(b0), full(w1), full(b1),
                  full(w2), full(b2)],
        out_specs=row,
        out_shape=jax.ShapeDtypeStruct((_EPAD, _L), jnp.float32),
    )(td, ts, ea, w0, b0, w1, b1, w2, b2)


def _update_body(h_ref, p_ref, wuh, wua, bu0, wu1, bu1,
                 wu2, bu2, gw, gb, o_ref):
    agg = p_ref[0] + p_ref[1]
    h = h_ref[...]
    u = _elu(_dot(h, wuh[...]) + _dot(agg, wua[...]) + bu0[...])
    u = _elu(_dot(u, wu1[...]) + bu1[...])
    u = _elu(_dot(u, wu2[...]) + bu2[...])
    lane = lax.broadcasted_iota(jnp.int32, (1, _L), 1)
    m0 = lane < (_L // 2)
    half = float(_L // 2)
    s1 = jnp.sum(jnp.where(m0, u, 0.0), axis=1, keepdims=True)
    s2 = jnp.sum(jnp.where(m0, 0.0, u), axis=1, keepdims=True)
    mu = jnp.where(m0, s1, s2) / half
    d = u - mu
    v1 = jnp.sum(jnp.where(m0, d * d, 0.0), axis=1, keepdims=True)
    v2 = jnp.sum(jnp.where(m0, 0.0, d * d), axis=1, keepdims=True)
    var = jnp.where(m0, v1, v2) / half
    o_ref[...] = d * lax.rsqrt(var + 1e-5) * gw[...] + gb[...]


def _update_call(h, p, wuh, wua, bu0, wu1, bu1, wu2, bu2, gw, gb):
    grid = (_NPAD // _NROWB,)
    row = pl.BlockSpec((_NROWB, _L), lambda i: (i, 0))
    pr = pl.BlockSpec((2, _NROWB, _L), lambda i: (0, i, 0))
    full = lambda a: pl.BlockSpec(a.shape, lambda i: (0, 0))
    return pl.pallas_call(
        _update_body,
        grid=grid,
        in_specs=[row, pr] + [full(a) for a in
                              (wuh, wua, bu0, wu1, bu1, wu2, bu2, gw, gb)],
        out_specs=row,
        out_shape=jax.ShapeDtypeStruct((_NPAD, _L), jnp.float32),
    )(h, p, wuh, wua, bu0, wu1, bu1, wu2, bu2, gw, gb)


_MESH = plsc.VectorSubcoreMesh(core_axis_name="c", subcore_axis_name="s")


def _wid():
    return lax.axis_index("s") * _NC + lax.axis_index("c")


@functools.partial(
    pl.kernel,
    out_type=[jax.ShapeDtypeStruct((_EPAD, _L), jnp.float32)] * 2,
    mesh=_MESH,
    scratch_types=[
        pltpu.VMEM((_BPW, _NB), jnp.int32),
        pltpu.VMEM((_BPW, _NB), jnp.int32),
        pltpu.VMEM((3, _NB, _L), jnp.float32),
        pltpu.VMEM((3, _NB, _L), jnp.float32),
        pltpu.SemaphoreType.DMA,
        pltpu.SemaphoreType.DMA,
        pltpu.SemaphoreType.DMA,
        pltpu.SemaphoreType.DMA,
        pltpu.SemaphoreType.DMA,
        pltpu.SemaphoreType.DMA,
    ],
)
def _gather2(h_hbm, dsti_hbm, srci_hbm, td_hbm, ts_hbm,
             dsti_v, srci_v, dbuf, sbuf, sg0, sg1, sg2, so0, so1, so2):
    w = _wid()
    pltpu.sync_copy(dsti_hbm.at[pl.ds(w * _BPW, _BPW)], dsti_v)
    pltpu.sync_copy(srci_hbm.at[pl.ds(w * _BPW, _BPW)], srci_v)
    base = w * _BPW
    sg = (sg0, sg1, sg2)
    so = (so0, so1, so2)

    def start_g(i, k):
        pltpu.async_copy(h_hbm.at[dsti_v.at[i]], dbuf.at[k], sg[k])
        pltpu.async_copy(h_hbm.at[srci_v.at[i]], sbuf.at[k], sg[k])

    def wait_g(i, k):
        pltpu.make_async_copy(h_hbm.at[dsti_v.at[i]], dbuf.at[k], sg[k]).wait()
        pltpu.make_async_copy(h_hbm.at[srci_v.at[i]], sbuf.at[k], sg[k]).wait()

    def start_out(i, k):
        pltpu.async_copy(dbuf.at[k], td_hbm.at[pl.ds((base + i) * _NB, _NB)], so[k])
        pltpu.async_copy(sbuf.at[k], ts_hbm.at[pl.ds((base + i) * _NB, _NB)], so[k])

    def wait_out(i, k):
        pltpu.make_async_copy(dbuf.at[k], td_hbm.at[pl.ds((base + i) * _NB, _NB)], so[k]).wait()
        pltpu.make_async_copy(sbuf.at[k], ts_hbm.at[pl.ds((base + i) * _NB, _NB)], so[k]).wait()

    def visit(i, k, kprev):
        @pl.when(jnp.logical_and(i >= 1, i + 2 < _BPW))
        def _():
            wait_out(i - 1, kprev)
            start_g(i + 2, kprev)

        wait_g(i, k)
        start_out(i, k)

    start_g(0, 0)
    start_g(1, 1)
    start_g(2, 2)

    def body(j, carry):
        i = 3 * j
        visit(i, 0, 2)
        visit(i + 1, 1, 0)
        visit(i + 2, 2, 1)
        return carry

    lax.fori_loop(0, _BPW // 3, body, 0)
    visit(_BPW - 2, 0, 2)
    visit(_BPW - 1, 1, 0)
    wait_out(_BPW - 3, 2)
    wait_out(_BPW - 2, 0)
    wait_out(_BPW - 1, 1)


@functools.partial(
    pl.kernel,
    out_type=jax.ShapeDtypeStruct((_NC, _NPAD, _L), jnp.float32),
    mesh=_MESH,
    scratch_types=[
        pltpu.VMEM((_BPW, _NB), jnp.int32),
        pltpu.VMEM((_NB, _L), jnp.float32),
        pltpu.VMEM((_NB, _L), jnp.float32),
        pltpu.VMEM_SHARED((_NPAD, _L), jnp.float32),
        pltpu.SemaphoreType.DMA,
        pltpu.SemaphoreType.DMA,
    ],
)
def _scatter_add(h2_hbm, dsti_hbm, p_hbm, dsti_v, r0, r1, acc, si0, si1):
    cid = lax.axis_index("c")
    sid = lax.axis_index("s")
    w = _wid()
    base = w * _BPW

    def zrow(r, c2):
        for jj in range(_L // 16):
            r0[r, pl.ds(jj * 16, 16)] = jnp.zeros((16,), jnp.float32)
        return c2

    lax.fori_loop(0, _NB, zrow, 0)

    def zcp(t, c2):
        pltpu.sync_copy(r0, acc.at[pl.ds(sid * _RPS + t * _NB, _NB)])
        return c2

    lax.fori_loop(0, _RPS // _NB, zcp, 0)
    pltpu.sync_copy(dsti_hbm.at[pl.ds(base, _BPW)], dsti_v)
    plsc.subcore_barrier()

    def in_of(i, rr, si):
        return pltpu.make_async_copy(
            h2_hbm.at[pl.ds((base + i) * _NB, _NB)], rr, si)

    pltpu.async_copy(h2_hbm.at[pl.ds(base * _NB, _NB)], r0, si0)
    pltpu.async_copy(h2_hbm.at[pl.ds((base + 1) * _NB, _NB)], r1, si1)

    def body(j, carry):
        i0 = 2 * j
        i1 = i0 + 1
        in_of(i0, r0, si0).wait()
        pltpu.sync_copy(r0, acc.at[dsti_v.at[i0]], add=True)

        @pl.when(j < _BPW // 2 - 1)
        def _():
            pltpu.async_copy(h2_hbm.at[pl.ds((base + i0 + 2) * _NB, _NB)], r0, si0)

        in_of(i1, r1, si1).wait()
        pltpu.sync_copy(r1, acc.at[dsti_v.at[i1]], add=True)

        @pl.when(j < _BPW // 2 - 1)
        def _():
            pltpu.async_copy(h2_hbm.at[pl.ds((base + i1 + 2) * _NB, _NB)], r1, si1)

        return carry

    lax.fori_loop(0, _BPW // 2, body, 0)
    plsc.subcore_barrier()
    pltpu.sync_copy(acc.at[pl.ds(sid * _RPS, _RPS)],
                    p_hbm.at[cid, pl.ds(sid * _RPS, _RPS)])


def kernel(x, edge_index, edge_attr, We0, be0, We1, be1, We2, be2,
           Wm0, bm0, Wm1, bm1, Wm2, bm2, Wu0, bu0, Wu1, bu1, Wu2, bu2,
           Wd0, bd0, Wd1, bd1, Wd2, bd2, gn_w, gn_b):
    r2 = lambda b: b.reshape(1, -1)
    src = edge_index[0]
    dst = edge_index[1]
    dst_p = jnp.pad(dst, (0, _EPAD - _E), constant_values=_N).reshape(_NBATCH, _NB)
    src_p = jnp.pad(src, (0, _EPAD - _E), constant_values=_N).reshape(_NBATCH, _NB)
    ea_p = jnp.pad(edge_attr, ((0, _EPAD - _E), (0, 0)))
    x_p = jnp.pad(x, ((0, _NPAD - _N), (0, 0)))

    h = _mlp3_call(x_p, We0, r2(be0), We1, r2(be1), We2, r2(be2), final_elu=True)

    for l in range(_NL):
        td, ts = _gather2(h, dst_p, src_p)
        m = _edge_call(td, ts, ea_p, Wm0[l], r2(bm0[l]), Wm1[l], r2(bm1[l]),
                       Wm2[l], r2(bm2[l]))
        p = _scatter_add(m, dst_p)
        h = _update_call(h, p,
                         Wu0[l][:_L], Wu0[l][_L:], r2(bu0[l]),
                         Wu1[l], r2(bu1[l]), Wu2[l], r2(bu2[l]),
                         r2(gn_w), r2(gn_b))

    wd2p = jnp.pad(Wd2, ((0, 0), (0, _L - Wd2.shape[1])))
    bd2p = jnp.pad(bd2, (0, _L - bd2.shape[0]))
    y = _mlp3_call(h, Wd0, r2(bd0), Wd1, r2(bd1), wd2p, r2(bd2p), final_elu=False)
    return y[:_N, :Wd2.shape[1]]


# speedup vs baseline: 1.3183x; 1.3183x over previous
import functools

import jax
import jax.numpy as jnp
from jax import lax
from jax.experimental import pallas as pl
from jax.experimental.pallas import tpu as pltpu
from jax.experimental.pallas import tpu_sc as plsc

_N = 10000
_E = 320000
_L = 128
_DE = 16
_NL = 4

_NPAD = 10240
_NB = 128
_EPAD = 327680
_NBATCH = _EPAD // _NB
_NC, _NS = 2, 16
_NW = _NC * _NS
_BPW = _NBATCH // _NW
_RPS = _NPAD // _NS

_NROWB = 1024
_EROWB = 2048


def _dot(a, b):
    return jax.lax.dot(a, b)


def _elu(x):
    return jnp.where(x > 0, x, jnp.exp(jnp.minimum(x, 0.0)) - 1.0)


def _mlp3_body(x_ref, w0, b0, w1, b1, w2, b2, o_ref, *, final_elu):
    h = _elu(_dot(x_ref[...], w0[...]) + b0[...])
    h = _elu(_dot(h, w1[...]) + b1[...])
    y = _dot(h, w2[...]) + b2[...]
    o_ref[...] = _elu(y) if final_elu else y


def _mlp3_call(x, w0, b0, w1, b1, w2, b2, final_elu):
    n = x.shape[0]
    grid = (n // _NROWB,)
    row = pl.BlockSpec((_NROWB, x.shape[1]), lambda i: (i, 0))
    full = lambda a: pl.BlockSpec(a.shape, lambda i: (0, 0))
    return pl.pallas_call(
        functools.partial(_mlp3_body, final_elu=final_elu),
        grid=grid,
        in_specs=[row, full(w0), full(b0), full(w1), full(b1), full(w2), full(b2)],
        out_specs=pl.BlockSpec((_NROWB, w2.shape[1]), lambda i: (i, 0)),
        out_shape=jax.ShapeDtypeStruct((n, w2.shape[1]), jnp.float32),
    )(x, w0, b0, w1, b1, w2, b2)


def _proj_body(h_ref, wd, ws, a_ref, b_ref):
    h = h_ref[...]
    a_ref[...] = _dot(h, wd[...])
    b_ref[...] = _dot(h, ws[...])


def _proj_call(h, wd, ws):
    grid = (_NPAD // _NROWB,)
    row = pl.BlockSpec((_NROWB, _L), lambda i: (i, 0))
    full = lambda a: pl.BlockSpec(a.shape, lambda i: (0, 0))
    return pl.pallas_call(
        _proj_body,
        grid=grid,
        in_specs=[row, full(wd), full(ws)],
        out_specs=[row, row],
        out_shape=[jax.ShapeDtypeStruct((_NPAD, _L), jnp.float32)] * 2,
    )(h, wd, ws)


def _edge_body(t_ref, ea_ref, we, b0, w1, b1, w2, b2, o_ref):
    h1 = _elu(t_ref[...] + _dot(ea_ref[...], we[...]) + b0[...])
    h2 = _elu(_dot(h1, w1[...]) + b1[...])
    o_ref[...] = _dot(h2, w2[...]) + b2[...]


def _edge_call(t, ea, we, b0, w1, b1, w2, b2):
    grid = (_EPAD // _EROWB,)
    row = pl.BlockSpec((_EROWB, _L), lambda i: (i, 0))
    ear = pl.BlockSpec((_EROWB, _DE), lambda i: (i, 0))
    full = lambda a: pl.BlockSpec(a.shape, lambda i: (0, 0))
    return pl.pallas_call(
        _edge_body,
        grid=grid,
        in_specs=[row, ear, full(we), full(b0), full(w1), full(b1), full(w2), full(b2)],
        out_specs=row,
        out_shape=jax.ShapeDtypeStruct((_EPAD, _L), jnp.float32),
    )(t, ea, we, b0, w1, b1, w2, b2)


def _update_body(h_ref, p_ref, wuh, wua, bu0, wu1, bu1, wu2, bu2, gw, gb, o_ref):
    agg = p_ref[0] + p_ref[1]
    h = h_ref[...]
    u = _elu(_dot(h, wuh[...]) + _dot(agg, wua[...]) + bu0[...])
    u = _elu(_dot(u, wu1[...]) + bu1[...])
    u = _elu(_dot(u, wu2[...]) + bu2[...])
    lane = lax.broadcasted_iota(jnp.int32, (1, _L), 1)
    m0 = lane < (_L // 2)
    half = float(_L // 2)
    s1 = jnp.sum(jnp.where(m0, u, 0.0), axis=1, keepdims=True)
    s2 = jnp.sum(jnp.where(m0, 0.0, u), axis=1, keepdims=True)
    mu = jnp.where(m0, s1, s2) / half
    d = u - mu
    v1 = jnp.sum(jnp.where(m0, d * d, 0.0), axis=1, keepdims=True)
    v2 = jnp.sum(jnp.where(m0, 0.0, d * d), axis=1, keepdims=True)
    var = jnp.where(m0, v1, v2) / half
    o_ref[...] = d * lax.rsqrt(var + 1e-5) * gw[...] + gb[...]


def _update_call(h, p, wuh, wua, bu0, wu1, bu1, wu2, bu2, gw, gb):
    grid = (_NPAD // _NROWB,)
    row = pl.BlockSpec((_NROWB, _L), lambda i: (i, 0))
    pr = pl.BlockSpec((2, _NROWB, _L), lambda i: (0, i, 0))
    full = lambda a: pl.BlockSpec(a.shape, lambda i: (0, 0))
    return pl.pallas_call(
        _update_body,
        grid=grid,
        in_specs=[row, pr] + [full(a) for a in
                              (wuh, wua, bu0, wu1, bu1, wu2, bu2, gw, gb)],
        out_specs=row,
        out_shape=jax.ShapeDtypeStruct((_NPAD, _L), jnp.float32),
    )(h, p, wuh, wua, bu0, wu1, bu1, wu2, bu2, gw, gb)


_MESH = plsc.VectorSubcoreMesh(core_axis_name="c", subcore_axis_name="s")


def _wid():
    return lax.axis_index("s") * _NC + lax.axis_index("c")


@functools.partial(
    pl.kernel,
    out_type=jax.ShapeDtypeStruct((_EPAD, _L), jnp.float32),
    mesh=_MESH,
    scratch_types=[
        pltpu.VMEM((_BPW, _NB), jnp.int32),
        pltpu.VMEM((_BPW, _NB), jnp.int32),
        pltpu.VMEM((3, _NB, _L), jnp.float32),
        pltpu.VMEM((3, _NB, _L), jnp.float32),
        pltpu.SemaphoreType.DMA,
        pltpu.SemaphoreType.DMA,
        pltpu.SemaphoreType.DMA,
        pltpu.SemaphoreType.DMA,
        pltpu.SemaphoreType.DMA,
        pltpu.SemaphoreType.DMA,
    ],
)
def _gather_add(a_hbm, b_hbm, dsti_hbm, srci_hbm, t_hbm,
                dsti_v, srci_v, abuf, bbuf, sg0, sg1, sg2, so0, so1, so2):
    w = _wid()
    pltpu.sync_copy(dsti_hbm.at[pl.ds(w * _BPW, _BPW)], dsti_v)
    pltpu.sync_copy(srci_hbm.at[pl.ds(w * _BPW, _BPW)], srci_v)
    base = w * _BPW
    sg = (sg0, sg1, sg2)
    so = (so0, so1, so2)

    def start_g(i, k):
        pltpu.async_copy(a_hbm.at[dsti_v.at[i]], abuf.at[k], sg[k])
        pltpu.async_copy(b_hbm.at[srci_v.at[i]], bbuf.at[k], sg[k])

    def wait_g(i, k):
        pltpu.make_async_copy(a_hbm.at[dsti_v.at[i]], abuf.at[k], sg[k]).wait()
        pltpu.make_async_copy(b_hbm.at[srci_v.at[i]], bbuf.at[k], sg[k]).wait()

    def start_out(i, k):
        pltpu.async_copy(abuf.at[k], t_hbm.at[pl.ds((base + i) * _NB, _NB)], so[k])

    def wait_out(i, k):
        pltpu.make_async_copy(
            abuf.at[k], t_hbm.at[pl.ds((base + i) * _NB, _NB)], so[k]).wait()

    def add(k):
        def add_rows(r, c2):
            for rr in range(2):
                for jj in range(_L // 16):
                    sl = pl.ds(jj * 16, 16)
                    plsc.addupdate(abuf.at[k, 2 * r + rr, sl],
                                   bbuf[k, 2 * r + rr, sl])
            return c2
        lax.fori_loop(0, _NB // 2, add_rows, 0)

    def visit(i, k, kprev):
        @pl.when(jnp.logical_and(i >= 1, i + 2 < _BPW))
        def _():
            wait_out(i - 1, kprev)
            start_g(i + 2, kprev)

        wait_g(i, k)
        add(k)
        start_out(i, k)

    start_g(0, 0)
    start_g(1, 1)
    start_g(2, 2)

    def body(j, carry):
        i = 3 * j
        visit(i, 0, 2)
        visit(i + 1, 1, 0)
        visit(i + 2, 2, 1)
        return carry

    lax.fori_loop(0, _BPW // 3, body, 0)
    visit(_BPW - 2, 0, 2)
    visit(_BPW - 1, 1, 0)
    wait_out(_BPW - 3, 2)
    wait_out(_BPW - 2, 0)
    wait_out(_BPW - 1, 1)


@functools.partial(
    pl.kernel,
    out_type=jax.ShapeDtypeStruct((_NC, _NPAD, _L), jnp.float32),
    mesh=_MESH,
    scratch_types=[
        pltpu.VMEM((_BPW, _NB), jnp.int32),
        pltpu.VMEM((_NB, _L), jnp.float32),
        pltpu.VMEM((_NB, _L), jnp.float32),
        pltpu.VMEM_SHARED((_NPAD, _L), jnp.float32),
        pltpu.SemaphoreType.DMA,
        pltpu.SemaphoreType.DMA,
    ],
)
def _scatter_add(h2_hbm, dsti_hbm, p_hbm, dsti_v, r0, r1, acc, si0, si1):
    cid = lax.axis_index("c")
    sid = lax.axis_index("s")
    w = _wid()
    base = w * _BPW

    def zrow(r, c2):
        for jj in range(_L // 16):
            r0[r, pl.ds(jj * 16, 16)] = jnp.zeros((16,), jnp.float32)
        return c2

    lax.fori_loop(0, _NB, zrow, 0)

    def zcp(t, c2):
        pltpu.sync_copy(r0, acc.at[pl.ds(sid * _RPS + t * _NB, _NB)])
        return c2

    lax.fori_loop(0, _RPS // _NB, zcp, 0)
    pltpu.sync_copy(dsti_hbm.at[pl.ds(base, _BPW)], dsti_v)
    plsc.subcore_barrier()

    def in_of(i, rr, si):
        return pltpu.make_async_copy(
            h2_hbm.at[pl.ds((base + i) * _NB, _NB)], rr, si)

    pltpu.async_copy(h2_hbm.at[pl.ds(base * _NB, _NB)], r0, si0)
    pltpu.async_copy(h2_hbm.at[pl.ds((base + 1) * _NB, _NB)], r1, si1)

    def body(j, carry):
        i0 = 2 * j
        i1 = i0 + 1
        in_of(i0, r0, si0).wait()
        pltpu.sync_copy(r0, acc.at[dsti_v.at[i0]], add=True)

        @pl.when(j < _BPW // 2 - 1)
        def _():
            pltpu.async_copy(h2_hbm.at[pl.ds((base + i0 + 2) * _NB, _NB)], r0, si0)

        in_of(i1, r1, si1).wait()
        pltpu.sync_copy(r1, acc.at[dsti_v.at[i1]], add=True)

        @pl.when(j < _BPW // 2 - 1)
        def _():
            pltpu.async_copy(h2_hbm.at[pl.ds((base + i1 + 2) * _NB, _NB)], r1, si1)

        return carry

    lax.fori_loop(0, _BPW // 2, body, 0)
    plsc.subcore_barrier()
    pltpu.sync_copy(acc.at[pl.ds(sid * _RPS, _RPS)],
                    p_hbm.at[cid, pl.ds(sid * _RPS, _RPS)])


def kernel(x, edge_index, edge_attr, We0, be0, We1, be1, We2, be2,
           Wm0, bm0, Wm1, bm1, Wm2, bm2, Wu0, bu0, Wu1, bu1, Wu2, bu2,
           Wd0, bd0, Wd1, bd1, Wd2, bd2, gn_w, gn_b):
    r2 = lambda b: b.reshape(1, -1)
    src = edge_index[0]
    dst = edge_index[1]
    dst_p = jnp.pad(dst, (0, _EPAD - _E), constant_values=_N).reshape(_NBATCH, _NB)
    src_p = jnp.pad(src, (0, _EPAD - _E), constant_values=_N).reshape(_NBATCH, _NB)
    ea_p = jnp.pad(edge_attr, ((0, _EPAD - _E), (0, 0)))
    x_p = jnp.pad(x, ((0, _NPAD - _N), (0, 0)))

    h = _mlp3_call(x_p, We0, r2(be0), We1, r2(be1), We2, r2(be2), final_elu=True)

    for l in range(_NL):
        a, b = _proj_call(h, Wm0[l][:_L], Wm0[l][_L:2 * _L])
        t = _gather_add(a, b, dst_p, src_p)
        m = _edge_call(t, ea_p, Wm0[l][2 * _L:], r2(bm0[l]), Wm1[l], r2(bm1[l]),
                       Wm2[l], r2(bm2[l]))
        p = _scatter_add(m, dst_p)
        h = _update_call(h, p,
                         Wu0[l][:_L], Wu0[l][_L:], r2(bu0[l]),
                         Wu1[l], r2(bu1[l]), Wu2[l], r2(bu2[l]),
                         r2(gn_w), r2(gn_b))

    wd2p = jnp.pad(Wd2, ((0, 0), (0, _L - Wd2.shape[1])))
    bd2p = jnp.pad(bd2, (0, _L - bd2.shape[0]))
    y = _mlp3_call(h, Wd0, r2(bd0), Wd1, r2(bd1), wd2p, r2(bd2p), final_elu=False)
    return y[:_N, :Wd2.shape[1]]
